# baseline (speedup 1.0000x reference)
import functools

import jax
import jax.numpy as jnp
from jax import lax
from jax.experimental import pallas as pl
from jax.experimental.pallas import tpu as pltpu
from jax.experimental.pallas import tpu_sc as plsc

_NC = 2
_NS = 16
_L = 16
_G = 2


def _onehot_body(B, Lx, D, x_hbm, out_hbm, xv, buf0, buf1, sem0, sem1):
    NW = _NC * _NS
    BPW = B // NW
    wid = lax.axis_index("s") * _NC + lax.axis_index("c")
    base = wid * BPW

    xcopy = pltpu.make_async_copy(x_hbm.at[pl.ds(base, BPW)], xv, sem0)
    xcopy.start()

    iota = lax.iota(jnp.int32, _L)
    ones_v = jnp.full((_L,), 1.0, jnp.float32)
    zeros_v = jnp.zeros((_L,), jnp.float32)
    n_full = Lx // _L
    tail = Lx - n_full * _L

    def scatter_batch(b, buf, val, g):
        for c in range(n_full):
            xk = xv[b, pl.ds(c * _L, _L)]
            plsc.store_scatter(buf, [g * Lx + c * _L + iota, xk], val)
        if tail:
            off = Lx - _L
            xk = xv[b, pl.ds(off, _L)]
            plsc.store_scatter(
                buf, [g * Lx + off + iota, xk], val, mask=iota >= (_L - tail)
            )

    def scatter_plane(i, buf, val):
        for g in range(_G):
            scatter_batch(i * _G + g, buf, val, g)

    NP = BPW // _G
    RP = _G * Lx
    rbase = base * Lx

    def dst(i):
        return out_hbm.at[pl.ds(rbase + i * RP, RP)]

    def zero_plane(buf):
        def zb(l, carry):
            for c in range(D // _L):
                buf[l, pl.ds(c * _L, _L)] = zeros_v
            return carry

        lax.fori_loop(0, RP, zb, 0)

    zero_plane(buf0)
    xcopy.wait()
    scatter_plane(0, buf0, ones_v)
    pltpu.make_async_copy(buf0, dst(0), sem0).start()
    zero_plane(buf1)

    def mbody(i, carry):
        def run(buf, sem):
            @pl.when(i >= 2)
            def _():
                pltpu.make_async_copy(buf, dst(i - 2), sem).wait()
                scatter_plane(i - 2, buf, zeros_v)

            scatter_plane(i, buf, ones_v)
            pltpu.make_async_copy(buf, dst(i), sem).start()

        @pl.when(i % 2 == 0)
        def _():
            run(buf0, sem0)

        @pl.when(i % 2 == 1)
        def _():
            run(buf1, sem1)

        return carry

    lax.fori_loop(1, NP, mbody, 0)

    pltpu.make_async_copy(buf0, dst(NP - 2), sem0).wait()
    pltpu.make_async_copy(buf1, dst(NP - 1), sem1).wait()


def kernel(X_in, ones):
    B = X_in.shape[0]
    Lx = X_in.shape[-1]
    D = ones.shape[0]
    BPW = B // (_NC * _NS)

    X = X_in.reshape(B, Lx).astype(jnp.int32)
    body = functools.partial(_onehot_body, B, Lx, D)
    out = pl.kernel(
        body,
        out_type=jax.ShapeDtypeStruct((B * Lx, D), jnp.float32),
        mesh=plsc.VectorSubcoreMesh(
            core_axis_name="c", subcore_axis_name="s",
            num_cores=_NC, num_subcores=_NS,
        ),
        compiler_params=pltpu.CompilerParams(needs_layout_passes=False),
        scratch_types=[
            pltpu.VMEM((BPW, Lx), jnp.int32),
            pltpu.VMEM((_G * Lx, D), jnp.float32),
            pltpu.VMEM((_G * Lx, D), jnp.float32),
            pltpu.SemaphoreType.DMA,
            pltpu.SemaphoreType.DMA,
        ],
    )(X)
    return jnp.transpose(out.reshape(B, Lx, D), (0, 2, 1))


# speedup vs baseline: 1.0207x; 1.0207x over previous
import functools

import jax
import jax.numpy as jnp
from jax import lax
from jax.experimental import pallas as pl
from jax.experimental.pallas import tpu as pltpu
from jax.experimental.pallas import tpu_sc as plsc

_NC = 2
_NS = 16
_L = 16


def _onehot_body(B, Lx, D, x_hbm, out_hbm, xv, buf0, buf1, sem0, sem1):
    NW = _NC * _NS
    BPW = B // NW
    wid = lax.axis_index("s") * _NC + lax.axis_index("c")
    base = wid * BPW

    xcopy = pltpu.make_async_copy(x_hbm.at[pl.ds(base, BPW)], xv, sem0)
    xcopy.start()

    iota = lax.iota(jnp.int32, _L)
    ones_v = jnp.full((_L,), 1.0, jnp.float32)
    zeros_v = jnp.zeros((_L,), jnp.float32)
    n_full = Lx // _L
    tail = Lx - n_full * _L

    def scatter_batch(b, buf, val):
        for c in range(n_full):
            xk = xv[b, pl.ds(c * _L, _L)]
            plsc.store_scatter(buf, [c * _L + iota, xk], val)
        if tail:
            off = Lx - _L
            xk = xv[b, pl.ds(off, _L)]
            plsc.store_scatter(
                buf, [off + iota, xk], val, mask=iota >= (_L - tail)
            )

    def zero_plane(buf):
        def zb(l, carry):
            for c in range(D // _L):
                buf[l, pl.ds(c * _L, _L)] = zeros_v
            return carry

        lax.fori_loop(0, Lx, zb, 0)

    zero_plane(buf0)
    xcopy.wait()
    scatter_batch(0, buf0, ones_v)
    pltpu.make_async_copy(buf0, out_hbm.at[base], sem0).start()
    zero_plane(buf1)
    scatter_batch(1, buf1, ones_v)
    pltpu.make_async_copy(buf1, out_hbm.at[base + 1], sem1).start()

    def mbody(k, carry):
        i0 = 2 * k
        for d, buf, sem in ((0, buf0, sem0), (1, buf1, sem1)):
            i = i0 + d
            pltpu.make_async_copy(buf, out_hbm.at[base + i - 2], sem).wait()
            scatter_batch(i - 2, buf, zeros_v)
            scatter_batch(i, buf, ones_v)
            pltpu.make_async_copy(buf, out_hbm.at[base + i], sem).start()
        return carry

    lax.fori_loop(1, BPW // 2, mbody, 0)

    pltpu.make_async_copy(buf0, out_hbm.at[base + BPW - 2], sem0).wait()
    pltpu.make_async_copy(buf1, out_hbm.at[base + BPW - 1], sem1).wait()


def kernel(X_in, ones):
    B = X_in.shape[0]
    Lx = X_in.shape[-1]
    D = ones.shape[0]
    BPW = B // (_NC * _NS)

    X = X_in.reshape(B, Lx).astype(jnp.int32)
    body = functools.partial(_onehot_body, B, Lx, D)
    out = pl.kernel(
        body,
        out_type=jax.ShapeDtypeStruct((B, Lx, D), jnp.float32),
        mesh=plsc.VectorSubcoreMesh(
            core_axis_name="c", subcore_axis_name="s",
            num_cores=_NC, num_subcores=_NS,
        ),
        compiler_params=pltpu.CompilerParams(needs_layout_passes=False),
        scratch_types=[
            pltpu.VMEM((BPW, Lx), jnp.int32),
            pltpu.VMEM((Lx, D), jnp.float32),
            pltpu.VMEM((Lx, D), jnp.float32),
            pltpu.SemaphoreType.DMA,
            pltpu.SemaphoreType.DMA,
        ],
    )(X)
    return jnp.transpose(out, (0, 2, 1))


# speedup vs baseline: 1.0207x; 1.0001x over previous
import functools

import jax
import jax.numpy as jnp
from jax import lax
from jax.experimental import pallas as pl
from jax.experimental.pallas import tpu as pltpu
from jax.experimental.pallas import tpu_sc as plsc

_NC = 2
_NS = 16
_L = 16


def _onehot_body(B, Lx, D, x_hbm, out_hbm, xv, buf0, buf1, sem0, sem1):
    NW = _NC * _NS
    BPW = B // NW
    wid = lax.axis_index("s") * _NC + lax.axis_index("c")
    base = wid * BPW

    xcopy = pltpu.make_async_copy(x_hbm.at[pl.ds(base, BPW)], xv, sem0)
    xcopy.start()

    iota = lax.iota(jnp.int32, _L)
    ones_v = jnp.full((_L,), 1.0, jnp.float32)
    zeros_v = jnp.zeros((_L,), jnp.float32)
    n_full = Lx // _L
    tail = Lx - n_full * _L

    def scatter_batch(b, buf, val):
        for c in range(n_full):
            xk = xv[b, pl.ds(c * _L, _L)]
            plsc.store_scatter(buf, [c * _L + iota, xk], val)
        if tail:
            off = Lx - _L
            xk = xv[b, pl.ds(off, _L)]
            plsc.store_scatter(
                buf, [off + iota, xk], val, mask=iota >= (_L - tail)
            )

    def zero_plane(buf):
        def zb(j, carry):
            for r in range(4):
                for c in range(D // _L):
                    buf[4 * j + r, pl.ds(c * _L, _L)] = zeros_v
            return carry

        lax.fori_loop(0, Lx // 4, zb, 0)

    zero_plane(buf0)
    xcopy.wait()
    scatter_batch(0, buf0, ones_v)
    pltpu.make_async_copy(buf0, out_hbm.at[base], sem0).start()
    zero_plane(buf1)
    scatter_batch(1, buf1, ones_v)
    pltpu.make_async_copy(buf1, out_hbm.at[base + 1], sem1).start()

    def mbody(k, carry):
        i0 = 2 * k
        for d, buf, sem in ((0, buf0, sem0), (1, buf1, sem1)):
            i = i0 + d
            pltpu.make_async_copy(buf, out_hbm.at[base + i - 2], sem).wait()
            scatter_batch(i - 2, buf, zeros_v)
            scatter_batch(i, buf, ones_v)
            pltpu.make_async_copy(buf, out_hbm.at[base + i], sem).start()
        return carry

    lax.fori_loop(1, BPW // 2, mbody, 0)

    pltpu.make_async_copy(buf0, out_hbm.at[base + BPW - 2], sem0).wait()
    pltpu.make_async_copy(buf1, out_hbm.at[base + BPW - 1], sem1).wait()


def kernel(X_in, ones):
    B = X_in.shape[0]
    Lx = X_in.shape[-1]
    D = ones.shape[0]
    BPW = B // (_NC * _NS)

    X = X_in.reshape(B, Lx).astype(jnp.int32)
    body = functools.partial(_onehot_body, B, Lx, D)
    out = pl.kernel(
        body,
        out_type=jax.ShapeDtypeStruct((B, Lx, D), jnp.float32),
        mesh=plsc.VectorSubcoreMesh(
            core_axis_name="c", subcore_axis_name="s",
            num_cores=_NC, num_subcores=_NS,
        ),
        compiler_params=pltpu.CompilerParams(needs_layout_passes=False),
        scratch_types=[
            pltpu.VMEM((BPW, Lx), jnp.int32),
            pltpu.VMEM((Lx, D), jnp.float32),
            pltpu.VMEM((Lx, D), jnp.float32),
            pltpu.SemaphoreType.DMA,
            pltpu.SemaphoreType.DMA,
        ],
    )(X)
    return jnp.transpose(out, (0, 2, 1))
